# baseline (speedup 1.0000x reference)
import jax
import jax.numpy as jnp
from jax.experimental import pallas as pl
from jax.experimental.pallas import tpu as pltpu


def _gcn_body(adj_ref, v_ref, w1_ref, wout_ref, bout_ref, out_ref, support_ref):
    r = pl.program_id(0)

    @pl.when(r == 0)
    def _init():
        support_ref[...] = jnp.dot(
            v_ref[...], w1_ref[...], preferred_element_type=jnp.float32
        )
        out_ref[...] = bout_ref[...]

    h = jnp.dot(adj_ref[...], support_ref[...], preferred_element_type=jnp.float32)
    x = jnp.sum(jax.nn.relu(h), axis=1, keepdims=True)
    out_ref[...] += jnp.sum(x * wout_ref[...], axis=0, keepdims=True)


def kernel(v, adj, W1, W_out, b_out):
    B, N, F = v.shape
    L = W_out.shape[1]
    H = W1.shape[1]

    v2 = v.reshape(N, F)
    adj2 = adj.reshape(N, N)
    bout2 = b_out.reshape(1, L)

    BR = 400
    if N % BR != 0:
        BR = 8
    grid = (N // BR,)

    out = pl.pallas_call(
        _gcn_body,
        grid=grid,
        in_specs=[
            pl.BlockSpec((BR, N), lambda r: (r, 0)),
            pl.BlockSpec((N, F), lambda r: (0, 0)),
            pl.BlockSpec((F, H), lambda r: (0, 0)),
            pl.BlockSpec((BR, L), lambda r: (r, 0)),
            pl.BlockSpec((1, L), lambda r: (0, 0)),
        ],
        out_specs=pl.BlockSpec((1, L), lambda r: (0, 0)),
        out_shape=jax.ShapeDtypeStruct((1, L), jnp.float32),
        scratch_shapes=[pltpu.VMEM((N, H), jnp.float32)],
    )(adj2, v2, W1, W_out, bout2)

    return out.reshape(B, L)
